# Initial kernel scaffold; baseline (speedup 1.0000x reference)
import jax
import jax.numpy as jnp
from jax.experimental import pallas as pl


def kernel(input_tok, input_tok_type, input_tok_pos, input_ent_tok, input_ent_tok_length, input_ent_mask_type, input_ent, input_ent_type, ent_candidates, word_emb, ent_emb, pos_emb, type_emb, mask_emb, fusion_w, fusion_b, ln_g, ln_b):
    raise NotImplementedError("write your pallas kernel here")


# speedup vs baseline: 5.5025x; 5.5025x over previous
import functools

import jax
import jax.numpy as jnp
from jax import lax
from jax.experimental import pallas as pl
from jax.experimental.pallas import tpu as pltpu
from jax.experimental.pallas import tpu_sc as plsc

EPS = 1e-12
NW = 32
H = 128

TOK_K, TOK_NC = 128, 50
ET_K, ET_NC = 80, 20
EE_K, EE_NC = 80, 20
CD_K, CD_NC = 64, 50


def _sc_body(tok_idx, et_idx, ee_idx, cd_idx, word, ent,
             tok_out, et_out, ee_out, cd_out,
             tok_ib, et_ib, ee_ib, cd_ib, buf_a, buf_b, acc,
             sga, sgb, swa, swb):
    wid = lax.axis_index("s") * 2 + lax.axis_index("c")

    def gather_task(table, idx_hbm, idx_buf, out, nchunks, k, rpw):
        pltpu.sync_copy(idx_hbm.at[wid], idx_buf)
        ba = buf_a.at[pl.ds(0, k)]
        bb = buf_b.at[pl.ds(0, k)]
        base = wid * rpw

        def g_start(j, buf, sem):
            pltpu.make_async_copy(table.at[idx_buf.at[j]], buf, sem).start()

        def g_wait(j, buf, sem):
            pltpu.make_async_copy(table.at[idx_buf.at[j]], buf, sem).wait()

        def w_start(j, buf, sem):
            pltpu.make_async_copy(buf, out.at[pl.ds(base + j * k, k)], sem).start()

        def w_wait(j, buf, sem):
            pltpu.make_async_copy(buf, out.at[pl.ds(base + j * k, k)], sem).wait()

        g_start(0, ba, sga)
        nt = nchunks // 2

        def body(t, carry):
            e = 2 * t

            @pl.when(t > 0)
            def _():
                w_wait(e - 1, bb, swb)

            g_start(e + 1, bb, sgb)
            g_wait(e, ba, sga)
            w_start(e, ba, swa)
            w_wait(e, ba, swa)

            @pl.when(t + 1 < nt)
            def _():
                g_start(e + 2, ba, sga)

            g_wait(e + 1, bb, sgb)
            w_start(e + 1, bb, swb)
            return carry

        lax.fori_loop(0, nt, body, 0)
        w_wait(nchunks - 1, bb, swb)

    def enttok_task():
        pltpu.sync_copy(et_idx.at[wid], et_ib)
        k = ET_K
        ba = buf_a.at[pl.ds(0, k)]
        bb = buf_b.at[pl.ds(0, k)]
        base = wid * (ET_NC * ET_K)

        def g_start(m, j, buf, sem):
            pltpu.make_async_copy(word.at[et_ib.at[m, j]], buf, sem).start()

        def g_wait(m, j, buf, sem):
            pltpu.make_async_copy(word.at[et_ib.at[m, j]], buf, sem).wait()

        def add_into_acc(buf):
            def body(i, carry):
                r = i * 8
                for rr in range(8):
                    for c in range(8):
                        plsc.addupdate(acc.at[r + rr, pl.ds(c * 16, 16)],
                                       buf[r + rr, pl.ds(c * 16, 16)])
                return carry
            lax.fori_loop(0, k // 8, body, 0)

        def body(j, carry):
            g_start(0, j, acc, sga)
            g_start(1, j, bb, sgb)
            g_wait(0, j, acc, sga)
            g_start(2, j, ba, sga)
            for m in range(1, 10):
                buf, sem = (ba, sga) if m % 2 == 0 else (bb, sgb)
                g_wait(m, j, buf, sem)
                add_into_acc(buf)
                if m + 2 <= 9:
                    g_start(m + 2, j, buf, sem)
            pltpu.sync_copy(acc, et_out.at[pl.ds(base + j * k, k)])
            return carry

        lax.fori_loop(0, ET_NC, body, 0)

    gather_task(word, tok_idx, tok_ib, tok_out, TOK_NC, TOK_K, TOK_NC * TOK_K)
    enttok_task()
    gather_task(ent, ee_idx, ee_ib, ee_out, EE_NC, EE_K, EE_NC * EE_K)
    gather_task(ent, cd_idx, cd_ib, cd_out, CD_NC, CD_K, CD_NC * CD_K)


def _sc_gather(tok_idx, et_idx, ee_idx, cd_idx, word_emb, ent_emb,
               bt, be, bc):
    f32 = jnp.float32
    run = pl.kernel(
        _sc_body,
        out_type=[
            jax.ShapeDtypeStruct((bt, H), f32),
            jax.ShapeDtypeStruct((be, H), f32),
            jax.ShapeDtypeStruct((be, H), f32),
            jax.ShapeDtypeStruct((bc, H), f32),
        ],
        mesh=plsc.VectorSubcoreMesh(core_axis_name="c", subcore_axis_name="s"),
        scratch_types=[
            pltpu.VMEM((TOK_NC, TOK_K), jnp.int32),
            pltpu.VMEM((10, ET_NC, ET_K), jnp.int32),
            pltpu.VMEM((EE_NC, EE_K), jnp.int32),
            pltpu.VMEM((CD_NC, CD_K), jnp.int32),
            pltpu.VMEM((128, H), f32),
            pltpu.VMEM((128, H), f32),
            pltpu.VMEM((ET_K, H), f32),
            pltpu.SemaphoreType.DMA,
            pltpu.SemaphoreType.DMA,
            pltpu.SemaphoreType.DMA,
            pltpu.SemaphoreType.DMA,
        ],
    )
    return run(tok_idx, et_idx, ee_idx, cd_idx, word_emb, ent_emb)


def _ln(x, g, b):
    m = jnp.mean(x, axis=-1, keepdims=True)
    v = jnp.mean((x - m) ** 2, axis=-1, keepdims=True)
    return (x - m) / jnp.sqrt(v + EPS) * g + b


_DN = (((0,), (0,)), ((), ()))


def _onehot_rows(idx_1xn, table_ref, width, blk):
    iota = lax.broadcasted_iota(jnp.int32, (width, blk), 0)
    oh = (iota == idx_1xn).astype(jnp.float32)
    return lax.dot_general(oh, table_ref[...], _DN,
                           preferred_element_type=jnp.float32,
                           precision=lax.Precision.HIGHEST)


def _tok_tc_body(rows_ref, pidx_ref, tidx_ref, pos_ref, typ_ref, g_ref, b_ref,
                 out_ref, *, blk):
    x = rows_ref[...]
    x = x + _onehot_rows(pidx_ref[0], pos_ref, 256, blk)
    x = x + _onehot_rows(tidx_ref[0], typ_ref, 16, blk)
    out_ref[...] = _ln(x, g_ref[...], b_ref[...])


def _ent_tc_body(ee_ref, es_ref, lenf_ref, mnz_ref, midx_ref, tidx_ref,
                 mask_ref, w1_ref, w2_ref, fb_ref, typ_ref, g_ref, b_ref,
                 out_ref, *, blk):
    g = g_ref[...]
    b = b_ref[...]
    et = es_ref[...] / lenf_ref[0]
    mnz = mnz_ref[0]
    mrows = _onehot_rows(midx_ref[0], mask_ref, 8, blk)
    et = mnz * mrows + (1.0 - mnz) * et
    x = lax.dot_general(ee_ref[...], w1_ref[...], (((1,), (0,)), ((), ())),
                        preferred_element_type=jnp.float32,
                        precision=lax.Precision.HIGHEST)
    x = x + lax.dot_general(et, w2_ref[...], (((1,), (0,)), ((), ())),
                            preferred_element_type=jnp.float32,
                            precision=lax.Precision.HIGHEST)
    x = x + fb_ref[...]
    x = 0.5 * x * (1.0 + lax.erf(x * (2.0 ** -0.5)))
    x = _ln(x, g, b)
    x = x + _onehot_rows(tidx_ref[0], typ_ref, 16, blk)
    out_ref[...] = _ln(x, g, b)


def kernel(input_tok, input_tok_type, input_tok_pos, input_ent_tok,
           input_ent_tok_length, input_ent_mask_type, input_ent,
           input_ent_type, ent_candidates, word_emb, ent_emb, pos_emb,
           type_emb, mask_emb, fusion_w, fusion_b, ln_g, ln_b):
    B, T = input_tok.shape
    _, E, M = input_ent_tok.shape
    _, C = ent_candidates.shape
    BT, BE, BC = B * T, B * E, B * C
    f32 = jnp.float32

    tok_idx = input_tok.reshape(NW, TOK_NC, TOK_K)
    et_idx = (input_ent_tok.reshape(NW, ET_NC, ET_K, M)
              .transpose(0, 3, 1, 2))
    ee_idx = input_ent.reshape(NW, EE_NC, EE_K)
    cd_idx = ent_candidates.reshape(NW, CD_NC, CD_K)

    tok_rows, et_sum, ee_rows, cd_rows = _sc_gather(
        tok_idx, et_idx, ee_idx, cd_idx, word_emb, ent_emb, BT, BE, BC)

    BLK = 512
    nb = BT // BLK
    pos256 = pos_emb[:256]
    typ16 = jnp.zeros((16, H), f32).at[:type_emb.shape[0]].set(type_emb)
    pidx = input_tok_pos.reshape(nb, 1, BLK)
    tidx = input_tok_type.reshape(nb, 1, BLK)
    g2 = ln_g.reshape(1, H)
    b2 = ln_b.reshape(1, H)

    tok_out = pl.pallas_call(
        functools.partial(_tok_tc_body, blk=BLK),
        grid=(nb,),
        in_specs=[
            pl.BlockSpec((BLK, H), lambda i: (i, 0)),
            pl.BlockSpec((1, 1, BLK), lambda i: (i, 0, 0)),
            pl.BlockSpec((1, 1, BLK), lambda i: (i, 0, 0)),
            pl.BlockSpec((256, H), lambda i: (0, 0)),
            pl.BlockSpec((16, H), lambda i: (0, 0)),
            pl.BlockSpec((1, H), lambda i: (0, 0)),
            pl.BlockSpec((1, H), lambda i: (0, 0)),
        ],
        out_specs=pl.BlockSpec((BLK, H), lambda i: (i, 0)),
        out_shape=jax.ShapeDtypeStruct((BT, H), f32),
    )(tok_rows, pidx, tidx, pos256, typ16, g2, b2)

    BLK2 = 512
    nb2 = BE // BLK2
    lenf = input_ent_tok_length.astype(f32).reshape(nb2, BLK2, 1)
    mnz = (input_ent_mask_type != 0).astype(f32).reshape(nb2, BLK2, 1)
    midx = input_ent_mask_type.reshape(nb2, 1, BLK2)
    etidx = input_ent_type.reshape(nb2, 1, BLK2)
    mask8 = jnp.zeros((8, H), f32).at[:mask_emb.shape[0]].set(mask_emb)
    w1 = fusion_w[:H]
    w2 = fusion_w[H:]
    fb2 = fusion_b.reshape(1, H)

    ent_out = pl.pallas_call(
        functools.partial(_ent_tc_body, blk=BLK2),
        grid=(nb2,),
        in_specs=[
            pl.BlockSpec((BLK2, H), lambda i: (i, 0)),
            pl.BlockSpec((BLK2, H), lambda i: (i, 0)),
            pl.BlockSpec((1, BLK2, 1), lambda i: (i, 0, 0)),
            pl.BlockSpec((1, BLK2, 1), lambda i: (i, 0, 0)),
            pl.BlockSpec((1, 1, BLK2), lambda i: (i, 0, 0)),
            pl.BlockSpec((1, 1, BLK2), lambda i: (i, 0, 0)),
            pl.BlockSpec((8, H), lambda i: (0, 0)),
            pl.BlockSpec((H, H), lambda i: (0, 0)),
            pl.BlockSpec((H, H), lambda i: (0, 0)),
            pl.BlockSpec((1, H), lambda i: (0, 0)),
            pl.BlockSpec((16, H), lambda i: (0, 0)),
            pl.BlockSpec((1, H), lambda i: (0, 0)),
            pl.BlockSpec((1, H), lambda i: (0, 0)),
        ],
        out_specs=pl.BlockSpec((BLK2, H), lambda i: (i, 0)),
        out_shape=jax.ShapeDtypeStruct((BE, H), f32),
    )(ee_rows, et_sum, lenf, mnz, midx, etidx, mask8, w1, w2, fb2,
      typ16, g2, b2)

    return (tok_out.reshape(B, T, H),
            ent_out.reshape(B, E, H),
            cd_rows.reshape(B, C, H))


# speedup vs baseline: 7.1960x; 1.3078x over previous
import functools

import jax
import jax.numpy as jnp
from jax import lax
from jax.experimental import pallas as pl
from jax.experimental.pallas import tpu as pltpu
from jax.experimental.pallas import tpu_sc as plsc

EPS = 1e-12
NW = 32
H = 128

TOK_K, TOK_NC = 128, 50
ET_K, ET_NC = 80, 20
EE_K, EE_NC = 80, 20
CD_K, CD_NC = 64, 50


def _gather_task(wid, table, idx_hbm, idx_buf, out, buf_a, buf_b,
                 sga, sgb, swa, swb, nchunks, k, rpw):
    pltpu.sync_copy(idx_hbm.at[wid], idx_buf)
    ba = buf_a.at[pl.ds(0, k)]
    bb = buf_b.at[pl.ds(0, k)]
    base = wid * rpw

    def g_start(j, buf, sem):
        pltpu.make_async_copy(table.at[idx_buf.at[j]], buf, sem).start()

    def g_wait(j, buf, sem):
        pltpu.make_async_copy(table.at[idx_buf.at[j]], buf, sem).wait()

    def w_start(j, buf, sem):
        pltpu.make_async_copy(buf, out.at[pl.ds(base + j * k, k)], sem).start()

    def w_wait(j, buf, sem):
        pltpu.make_async_copy(buf, out.at[pl.ds(base + j * k, k)], sem).wait()

    g_start(0, ba, sga)
    nt = nchunks // 2

    def body(t, carry):
        e = 2 * t

        @pl.when(t > 0)
        def _():
            w_wait(e - 1, bb, swb)

        g_start(e + 1, bb, sgb)
        g_wait(e, ba, sga)
        w_start(e, ba, swa)
        w_wait(e, ba, swa)

        @pl.when(t + 1 < nt)
        def _():
            g_start(e + 2, ba, sga)

        g_wait(e + 1, bb, sgb)
        w_start(e + 1, bb, swb)
        return carry

    lax.fori_loop(0, nt, body, 0)
    w_wait(nchunks - 1, bb, swb)


def _sc_tok_body(tok_idx, word, tok_out,
                 tok_ib, buf_a, buf_b, sga, sgb, swa, swb):
    wid = lax.axis_index("s") * 2 + lax.axis_index("c")
    _gather_task(wid, word, tok_idx, tok_ib, tok_out, buf_a, buf_b,
                 sga, sgb, swa, swb, TOK_NC, TOK_K, TOK_NC * TOK_K)


def _sc_ent_body(et_idx, ee_idx, word, ent, et_out, ee_out,
                 et_ib, ee_ib, buf_a, buf_b, acc, sga, sgb, swa, swb):
    wid = lax.axis_index("s") * 2 + lax.axis_index("c")

    pltpu.sync_copy(et_idx.at[wid], et_ib)
    k = ET_K
    ba = buf_a.at[pl.ds(0, k)]
    bb = buf_b.at[pl.ds(0, k)]
    base = wid * (ET_NC * ET_K)

    def g_start(m, j, buf, sem):
        pltpu.make_async_copy(word.at[et_ib.at[m, j]], buf, sem).start()

    def g_wait(m, j, buf, sem):
        pltpu.make_async_copy(word.at[et_ib.at[m, j]], buf, sem).wait()

    def add_into_acc(buf):
        def abody(i, carry):
            r = i * 8
            for rr in range(8):
                for c in range(8):
                    plsc.addupdate(acc.at[r + rr, pl.ds(c * 16, 16)],
                                   buf[r + rr, pl.ds(c * 16, 16)])
            return carry
        lax.fori_loop(0, k // 8, abody, 0)

    def body(j, carry):
        g_start(0, j, acc, sga)
        g_start(1, j, bb, sgb)
        g_wait(0, j, acc, sga)
        g_start(2, j, ba, sga)
        for m in range(1, 10):
            buf, sem = (ba, sga) if m % 2 == 0 else (bb, sgb)
            g_wait(m, j, buf, sem)
            add_into_acc(buf)
            if m + 2 <= 9:
                g_start(m + 2, j, buf, sem)
        pltpu.sync_copy(acc, et_out.at[pl.ds(base + j * k, k)])
        return carry

    lax.fori_loop(0, ET_NC, body, 0)

    _gather_task(wid, ent, ee_idx, ee_ib, ee_out, buf_a, buf_b,
                 sga, sgb, swa, swb, EE_NC, EE_K, EE_NC * EE_K)


def _sc_cd_body(cd_idx, ent, cd_out,
                cd_ib, buf_a, buf_b, sga, sgb, swa, swb):
    wid = lax.axis_index("s") * 2 + lax.axis_index("c")
    _gather_task(wid, ent, cd_idx, cd_ib, cd_out, buf_a, buf_b,
                 sga, sgb, swa, swb, CD_NC, CD_K, CD_NC * CD_K)


_MESH = dict(core_axis_name="c", subcore_axis_name="s")


def _sems(n):
    return [pltpu.SemaphoreType.DMA] * n


def _ln(x, g, b):
    m = jnp.mean(x, axis=-1, keepdims=True)
    v = jnp.mean((x - m) ** 2, axis=-1, keepdims=True)
    return (x - m) / jnp.sqrt(v + EPS) * g + b


_DN = (((0,), (0,)), ((), ()))
_DNR = (((1,), (0,)), ((), ()))


def _onehot_rows(idx_1xn, table_ref, width, blk):
    iota = lax.broadcasted_iota(jnp.int32, (width, blk), 0)
    oh = (iota == idx_1xn).astype(jnp.bfloat16)
    return lax.dot_general(oh, table_ref[...], _DN,
                           preferred_element_type=jnp.float32)


def _tok_tc_body(rows_ref, pidx_ref, tidx_ref, pos_ref, typ_ref, g_ref, b_ref,
                 out_ref, *, blk):
    x = rows_ref[...]
    x = x + _onehot_rows(pidx_ref[0], pos_ref, 256, blk)
    x = x + _onehot_rows(tidx_ref[0], typ_ref, 16, blk)
    out_ref[...] = _ln(x, g_ref[...], b_ref[...])


def _ent_tc_body(ee_ref, es_ref, lenf_ref, mnz_ref, midx_ref, tidx_ref,
                 mask_ref, w1_ref, w2_ref, fb_ref, typ_ref, g_ref, b_ref,
                 out_ref, *, blk):
    g = g_ref[...]
    b = b_ref[...]
    et = es_ref[...] / lenf_ref[0]
    mnz = mnz_ref[0]
    mrows = _onehot_rows(midx_ref[0], mask_ref, 8, blk)
    et = mnz * mrows + (1.0 - mnz) * et
    x = lax.dot_general(ee_ref[...].astype(jnp.bfloat16), w1_ref[...], _DNR,
                        preferred_element_type=jnp.float32)
    x = x + lax.dot_general(et.astype(jnp.bfloat16), w2_ref[...], _DNR,
                            preferred_element_type=jnp.float32)
    x = x + fb_ref[...]
    x = 0.5 * x * (1.0 + lax.erf(x * (2.0 ** -0.5)))
    x = _ln(x, g, b)
    x = x + _onehot_rows(tidx_ref[0], typ_ref, 16, blk)
    out_ref[...] = _ln(x, g, b)


def _sc_tok(tok_idx, word_emb, bt):
    return pl.kernel(
        _sc_tok_body,
        out_type=jax.ShapeDtypeStruct((bt, H), jnp.float32),
        mesh=plsc.VectorSubcoreMesh(**_MESH),
        scratch_types=[
            pltpu.VMEM((TOK_NC, TOK_K), jnp.int32),
            pltpu.VMEM((128, H), jnp.float32),
            pltpu.VMEM((128, H), jnp.float32),
        ] + _sems(4),
    )(tok_idx, word_emb)


def _sc_ent(et_idx, ee_idx, word_emb, ent_emb, be):
    return pl.kernel(
        _sc_ent_body,
        out_type=[jax.ShapeDtypeStruct((be, H), jnp.float32),
                  jax.ShapeDtypeStruct((be, H), jnp.float32)],
        mesh=plsc.VectorSubcoreMesh(**_MESH),
        scratch_types=[
            pltpu.VMEM((10, ET_NC, ET_K), jnp.int32),
            pltpu.VMEM((EE_NC, EE_K), jnp.int32),
            pltpu.VMEM((128, H), jnp.float32),
            pltpu.VMEM((128, H), jnp.float32),
            pltpu.VMEM((ET_K, H), jnp.float32),
        ] + _sems(4),
    )(et_idx, ee_idx, word_emb, ent_emb)


def _sc_cd(cd_idx, ent_emb, bc):
    return pl.kernel(
        _sc_cd_body,
        out_type=jax.ShapeDtypeStruct((bc, H), jnp.float32),
        mesh=plsc.VectorSubcoreMesh(**_MESH),
        scratch_types=[
            pltpu.VMEM((CD_NC, CD_K), jnp.int32),
            pltpu.VMEM((128, H), jnp.float32),
            pltpu.VMEM((128, H), jnp.float32),
        ] + _sems(4),
    )(cd_idx, ent_emb)


def kernel(input_tok, input_tok_type, input_tok_pos, input_ent_tok,
           input_ent_tok_length, input_ent_mask_type, input_ent,
           input_ent_type, ent_candidates, word_emb, ent_emb, pos_emb,
           type_emb, mask_emb, fusion_w, fusion_b, ln_g, ln_b):
    B, T = input_tok.shape
    _, E, M = input_ent_tok.shape
    _, C = ent_candidates.shape
    BT, BE, BC = B * T, B * E, B * C
    f32 = jnp.float32
    bf16 = jnp.bfloat16

    tok_idx = input_tok.reshape(NW, TOK_NC, TOK_K)
    et_idx = (input_ent_tok.reshape(NW, ET_NC, ET_K, M)
              .transpose(0, 3, 1, 2))
    ee_idx = input_ent.reshape(NW, EE_NC, EE_K)
    cd_idx = ent_candidates.reshape(NW, CD_NC, CD_K)

    tok_rows = _sc_tok(tok_idx, word_emb, BT)
    et_sum, ee_rows = _sc_ent(et_idx, ee_idx, word_emb, ent_emb, BE)
    cd_rows = _sc_cd(cd_idx, ent_emb, BC)

    BLK = 512
    nb = BT // BLK
    pos256 = pos_emb[:256].astype(bf16)
    typ16 = jnp.zeros((16, H), f32).at[:type_emb.shape[0]].set(type_emb)
    typ16 = typ16.astype(bf16)
    pidx = input_tok_pos.reshape(nb, 1, BLK)
    tidx = input_tok_type.reshape(nb, 1, BLK)
    g2 = ln_g.reshape(1, H)
    b2 = ln_b.reshape(1, H)

    tok_out = pl.pallas_call(
        functools.partial(_tok_tc_body, blk=BLK),
        grid=(nb,),
        in_specs=[
            pl.BlockSpec((BLK, H), lambda i: (i, 0)),
            pl.BlockSpec((1, 1, BLK), lambda i: (i, 0, 0)),
            pl.BlockSpec((1, 1, BLK), lambda i: (i, 0, 0)),
            pl.BlockSpec((256, H), lambda i: (0, 0)),
            pl.BlockSpec((16, H), lambda i: (0, 0)),
            pl.BlockSpec((1, H), lambda i: (0, 0)),
            pl.BlockSpec((1, H), lambda i: (0, 0)),
        ],
        out_specs=pl.BlockSpec((BLK, H), lambda i: (i, 0)),
        out_shape=jax.ShapeDtypeStruct((BT, H), f32),
    )(tok_rows, pidx, tidx, pos256, typ16, g2, b2)

    BLK2 = 512
    nb2 = BE // BLK2
    lenf = input_ent_tok_length.astype(f32).reshape(nb2, BLK2, 1)
    mnz = (input_ent_mask_type != 0).astype(f32).reshape(nb2, BLK2, 1)
    midx = input_ent_mask_type.reshape(nb2, 1, BLK2)
    etidx = input_ent_type.reshape(nb2, 1, BLK2)
    mask8 = jnp.zeros((8, H), f32).at[:mask_emb.shape[0]].set(mask_emb)
    mask8 = mask8.astype(bf16)
    w1 = fusion_w[:H].astype(bf16)
    w2 = fusion_w[H:].astype(bf16)
    fb2 = fusion_b.reshape(1, H)

    ent_out = pl.pallas_call(
        functools.partial(_ent_tc_body, blk=BLK2),
        grid=(nb2,),
        in_specs=[
            pl.BlockSpec((BLK2, H), lambda i: (i, 0)),
            pl.BlockSpec((BLK2, H), lambda i: (i, 0)),
            pl.BlockSpec((1, BLK2, 1), lambda i: (i, 0, 0)),
            pl.BlockSpec((1, BLK2, 1), lambda i: (i, 0, 0)),
            pl.BlockSpec((1, 1, BLK2), lambda i: (i, 0, 0)),
            pl.BlockSpec((1, 1, BLK2), lambda i: (i, 0, 0)),
            pl.BlockSpec((8, H), lambda i: (0, 0)),
            pl.BlockSpec((H, H), lambda i: (0, 0)),
            pl.BlockSpec((H, H), lambda i: (0, 0)),
            pl.BlockSpec((1, H), lambda i: (0, 0)),
            pl.BlockSpec((16, H), lambda i: (0, 0)),
            pl.BlockSpec((1, H), lambda i: (0, 0)),
            pl.BlockSpec((1, H), lambda i: (0, 0)),
        ],
        out_specs=pl.BlockSpec((BLK2, H), lambda i: (i, 0)),
        out_shape=jax.ShapeDtypeStruct((BE, H), f32),
    )(ee_rows, et_sum, lenf, mnz, midx, etidx, mask8, w1, w2, fb2,
      typ16, g2, b2)

    return (tok_out.reshape(B, T, H),
            ent_out.reshape(B, E, H),
            cd_rows.reshape(B, C, H))
